# baseline (speedup 1.0000x reference)
import functools
import numpy as np
import jax
import jax.numpy as jnp
from jax import lax
from jax.experimental import pallas as pl
from jax.experimental.pallas import tpu as pltpu
from jax.experimental.pallas import tpu_sc as plsc

_C = 32
_K = _C * _C
_V = _C * (_C + 1) // 2
_G = _V // 16
_NW = 32
_CHUNK = 32


def _tril_flat():
    row, col = np.tril_indices(_C)
    return (row * _C + col).astype(np.int32)


def _sc_kernel(m_total):
    per_w = m_total // _NW
    n_chunks = per_w // _CHUNK
    mesh = plsc.VectorSubcoreMesh(core_axis_name="c", subcore_axis_name="s")

    @functools.partial(
        pl.kernel,
        mesh=mesh,
        out_type=jax.ShapeDtypeStruct((m_total * _V,), jnp.float32),
        compiler_params=pltpu.CompilerParams(needs_layout_passes=False),
        scratch_types=[
            pltpu.VMEM((_V,), jnp.int32),
            pltpu.VMEM((_CHUNK * _K,), jnp.float32),
            pltpu.VMEM((_CHUNK * _V,), jnp.float32),
        ],
    )
    def k(x_hbm, idx_hbm, out_hbm, idx_v, in_v, out_v):
        wid = lax.axis_index("s") * 2 + lax.axis_index("c")
        base = wid * per_w
        pltpu.sync_copy(idx_hbm, idx_v)

        def chunk_body(ci, carry):
            mbase = base + ci * _CHUNK
            pltpu.sync_copy(x_hbm.at[pl.ds(mbase * _K, _CHUNK * _K)], in_v)

            def mat_body(m, c2):
                off_in = jnp.full((16,), m * _K, jnp.int32)
                obase = m * _V
                for j in range(_G):
                    idxv = idx_v[pl.ds(j * 16, 16)] + off_in
                    vals = plsc.load_gather(in_v, [idxv])
                    out_v[pl.ds(obase + j * 16, 16)] = vals
                return c2

            lax.fori_loop(0, _CHUNK, mat_body, 0)
            pltpu.sync_copy(out_v, out_hbm.at[pl.ds(mbase * _V, _CHUNK * _V)])
            return carry

        lax.fori_loop(0, n_chunks, chunk_body, 0)

    return k


def kernel(inputs):
    T, N, B, C, C2 = inputs.shape
    M = T * N * B
    x = inputs.reshape(M * C * C2)
    idx = jnp.asarray(_tril_flat())
    out = _sc_kernel(M)(x, idx)
    return out.reshape(T, N, B, _V)


# speedup vs baseline: 1.3559x; 1.3559x over previous
import functools
import numpy as np
import jax
import jax.numpy as jnp
from jax import lax
from jax.experimental import pallas as pl
from jax.experimental.pallas import tpu as pltpu
from jax.experimental.pallas import tpu_sc as plsc

_C = 32
_K = _C * _C
_V = _C * (_C + 1) // 2
_G = _V // 16
_NW = 32
_CHUNK = 32


def _tril_flat():
    row, col = np.tril_indices(_C)
    return (row * _C + col).astype(np.int32)


def _sc_kernel(m_total):
    per_w = m_total // _NW
    n_chunks = per_w // _CHUNK
    assert n_chunks % 2 == 0
    mesh = plsc.VectorSubcoreMesh(core_axis_name="c", subcore_axis_name="s")

    @functools.partial(
        pl.kernel,
        mesh=mesh,
        out_type=jax.ShapeDtypeStruct((m_total * _V,), jnp.float32),
        compiler_params=pltpu.CompilerParams(needs_layout_passes=False),
        scratch_types=[
            pltpu.VMEM((_V,), jnp.int32),
            pltpu.VMEM((_CHUNK * _K,), jnp.float32),
            pltpu.VMEM((_CHUNK * _K,), jnp.float32),
            pltpu.VMEM((_CHUNK * _V,), jnp.float32),
            pltpu.VMEM((_CHUNK * _V,), jnp.float32),
            pltpu.SemaphoreType.DMA,
            pltpu.SemaphoreType.DMA,
            pltpu.SemaphoreType.DMA,
            pltpu.SemaphoreType.DMA,
        ],
    )
    def k(x_hbm, idx_hbm, out_hbm, idx_v, in0, in1, out0, out1,
          si0, si1, so0, so1):
        wid = lax.axis_index("s") * 2 + lax.axis_index("c")
        base = wid * per_w
        pltpu.sync_copy(idx_hbm, idx_v)
        tabs = [idx_v[pl.ds(j * 16, 16)] for j in range(_G)]
        ins, outs = [in0, in1], [out0, out1]
        sis, sos = [si0, si1], [so0, so1]

        def in_slice(ci):
            return x_hbm.at[pl.ds((base + ci * _CHUNK) * _K, _CHUNK * _K)]

        def out_slice(ci):
            return out_hbm.at[pl.ds((base + ci * _CHUNK) * _V, _CHUNK * _V)]

        pltpu.async_copy(in_slice(0), in0, si0)
        pltpu.async_copy(in_slice(1), in1, si1)

        def gloop(g, carry):
            for b in range(2):
                ci = g * 2 + b
                pltpu.make_async_copy(in_slice(ci), ins[b], sis[b]).wait()

                @pl.when(g >= 1)
                def _():
                    pltpu.make_async_copy(outs[b], out_slice(ci),
                                          sos[b]).wait()

                def mat_body(m, c2, b=b):
                    src = ins[b].at[pl.ds(m * _K, _K)]
                    ob = m * _V
                    for j in range(_G):
                        vals = plsc.load_gather(src, [tabs[j]])
                        outs[b][pl.ds(ob + j * 16, 16)] = vals
                    return c2

                lax.fori_loop(0, _CHUNK, mat_body, 0)
                pltpu.async_copy(outs[b], out_slice(ci), sos[b])

                @pl.when(g < n_chunks // 2 - 1)
                def _():
                    pltpu.async_copy(in_slice(ci + 2), ins[b], sis[b])
            return carry

        lax.fori_loop(0, n_chunks // 2, gloop, 0)
        pltpu.make_async_copy(out0, out_slice(0), so0).wait()
        pltpu.make_async_copy(out1, out_slice(1), so1).wait()

    return k


def kernel(inputs):
    T, N, B, C, C2 = inputs.shape
    M = T * N * B
    x = inputs.reshape(M * C * C2)
    idx = jnp.asarray(_tril_flat())
    out = _sc_kernel(M)(x, idx)
    return out.reshape(T, N, B, _V)


# speedup vs baseline: 1.5083x; 1.1124x over previous
import functools
import numpy as np
import jax
import jax.numpy as jnp
from jax import lax
from jax.experimental import pallas as pl
from jax.experimental.pallas import tpu as pltpu
from jax.experimental.pallas import tpu_sc as plsc

_C = 32
_K = _C * _C
_V = _C * (_C + 1) // 2
_G = _V // 16
_NW = 32
_CHUNK = 16
_NB = 4


def _tril_flat():
    row, col = np.tril_indices(_C)
    return (row * _C + col).astype(np.int32)


def _sc_kernel(m_total):
    per_w = m_total // _NW
    n_chunks = per_w // _CHUNK
    assert n_chunks % _NB == 0
    mesh = plsc.VectorSubcoreMesh(core_axis_name="c", subcore_axis_name="s")

    @functools.partial(
        pl.kernel,
        mesh=mesh,
        out_type=jax.ShapeDtypeStruct((m_total, _V), jnp.float32),
        compiler_params=pltpu.CompilerParams(
            needs_layout_passes=False, use_tc_tiling_on_sc=False),
        scratch_types=(
            [pltpu.VMEM((_V,), jnp.int32)]
            + [pltpu.VMEM((_CHUNK, _K), jnp.float32) for _ in range(_NB)]
            + [pltpu.VMEM((_CHUNK, _V), jnp.float32) for _ in range(_NB)]
            + [pltpu.VMEM((_CHUNK,), jnp.int32) for _ in range(2 * _NB)]
            + [pltpu.SemaphoreType.DMA for _ in range(2 * _NB)]
        ),
    )
    def k(x_hbm, idx_hbm, out_hbm, idx_v, *refs):
        ins = list(refs[0:_NB])
        outs = list(refs[_NB:2 * _NB])
        mins = list(refs[2 * _NB:3 * _NB])
        mouts = list(refs[3 * _NB:4 * _NB])
        sis = list(refs[4 * _NB:5 * _NB])
        sos = list(refs[5 * _NB:6 * _NB])
        wid = lax.axis_index("s") * 2 + lax.axis_index("c")
        base = wid * per_w
        pltpu.sync_copy(idx_hbm, idx_v)
        tabs = [idx_v[pl.ds(j * 16, 16)] for j in range(_G)]
        lane = lax.iota(jnp.int32, 16)

        def set_ids(ref, ci):
            start = base + ci * _CHUNK
            ref[pl.ds(0, 16)] = lane + start

        for b in range(_NB):
            set_ids(mins[b], b)
            pltpu.async_copy(x_hbm.at[mins[b]], ins[b], sis[b])

        def gloop(g, carry):
            for b in range(_NB):
                ci = g * _NB + b
                pltpu.make_async_copy(x_hbm.at[mins[b]], ins[b],
                                      sis[b]).wait()

                @pl.when(g >= 1)
                def _():
                    pltpu.make_async_copy(outs[b], out_hbm.at[mouts[b]],
                                          sos[b]).wait()

                set_ids(mouts[b], ci)

                def mat_body(m, c2, b=b):
                    row = jnp.full((16,), m, jnp.int32)
                    for w in range(0, _G, 8):
                        hi = min(w + 8, _G)
                        vals = [plsc.load_gather(ins[b], [row, tabs[j]])
                                for j in range(w, hi)]
                        for i, j in enumerate(range(w, hi)):
                            outs[b][m, pl.ds(j * 16, 16)] = vals[i]
                    return c2

                lax.fori_loop(0, _CHUNK, mat_body, 0)
                pltpu.async_copy(outs[b], out_hbm.at[mouts[b]], sos[b])

                @pl.when(g < n_chunks // _NB - 1)
                def _():
                    set_ids(mins[b], ci + _NB)
                    pltpu.async_copy(x_hbm.at[mins[b]], ins[b], sis[b])
            return carry

        lax.fori_loop(0, n_chunks // _NB, gloop, 0)
        for b in range(_NB):
            pltpu.make_async_copy(outs[b], out_hbm.at[mouts[b]],
                                  sos[b]).wait()

    return k


def kernel(inputs):
    T, N, B, C, C2 = inputs.shape
    M = T * N * B
    x = inputs.reshape(M, C * C2)
    idx = jnp.asarray(_tril_flat())
    out = _sc_kernel(M)(x, idx)
    return out.reshape(T, N, B, _V)
